# baseline (speedup 1.0000x reference)
import functools

import jax
import jax.numpy as jnp
from jax import lax
from jax.experimental import pallas as pl
from jax.experimental.pallas import tpu as pltpu
from jax.experimental.pallas import tpu_sc as plsc

N = 10000
NPAD = 10240
D = 128
E = 320000
NC = 2
NS = 16
NW = NC * NS
EPW = E // NW
CH = 128
EPWP = 10240
NCHUNK = EPWP // CH
RPS = NPAD // NS
LANES = 16
DCH = 80
DNCH = EPW // DCH


NGRP = 5
CPG = NCHUNK // NGRP


def _make_sc_agg():
    mesh = plsc.VectorSubcoreMesh(core_axis_name="c", subcore_axis_name="s")
    out_type = jax.ShapeDtypeStruct((NC, NPAD, D), jnp.float32)
    scratch = [
        pltpu.VMEM((CPG, CH), jnp.int32),
        pltpu.VMEM((CPG, CH), jnp.int32),
        pltpu.VMEM((CH, D), jnp.float32),
        pltpu.VMEM((CH, D), jnp.float32),
        pltpu.VMEM_SHARED((NPAD, D), jnp.float32),
        pltpu.SemaphoreType.DMA,
        pltpu.SemaphoreType.DMA,
        pltpu.SemaphoreType.DMA,
        pltpu.SemaphoreType.DMA,
    ]

    def body(h_hbm, src_hbm, dst_hbm, agg_hbm,
             src_v, dst_v, buf_a, buf_b, acc_sh, sem_a, sem_b, sem_sa, sem_sb):
        cid = lax.axis_index("c")
        sid = lax.axis_index("s")
        wid = cid * NS + sid

        zvec = jnp.zeros((LANES,), jnp.float32)

        def zrow(i, carry):
            r = i // (D // LANES)
            c = (i % (D // LANES)) * LANES
            buf_a[r, pl.ds(c, LANES)] = zvec
            return carry

        lax.fori_loop(0, CH * (D // LANES), zrow, 0)
        for t in range(RPS // CH):
            pltpu.sync_copy(buf_a, acc_sh.at[pl.ds(sid * RPS + t * CH, CH)])

        plsc.subcore_barrier()

        def gather(j, buf, sem):
            return pltpu.async_copy(h_hbm.at[src_v.at[j]], buf, sem)

        def gwait(buf, sem):
            pltpu.make_async_copy(h_hbm.at[src_v.at[0]], buf, sem).wait()

        def ascat(j, buf, sem):
            pltpu.async_copy(buf, acc_sh.at[dst_v.at[j]], sem, add=True)

        def swait(buf, sem):
            pltpu.make_async_copy(buf, acc_sh.at[dst_v.at[0]], sem).wait()

        for g in range(NGRP):
            pltpu.sync_copy(src_hbm.at[wid].at[g], src_v)
            pltpu.sync_copy(dst_hbm.at[wid].at[g], dst_v)
            gather(0, buf_a, sem_a)
            gather(1, buf_b, sem_b)

            def pair(p, carry):
                j = 2 * p
                gwait(buf_a, sem_a)
                ascat(j, buf_a, sem_sa)
                gwait(buf_b, sem_b)
                ascat(j + 1, buf_b, sem_sb)
                swait(buf_a, sem_sa)
                gather(j + 2, buf_a, sem_a)
                swait(buf_b, sem_sb)
                gather(j + 3, buf_b, sem_b)
                return carry

            lax.fori_loop(0, CPG // 2 - 1, pair, 0)
            gwait(buf_a, sem_a)
            ascat(CPG - 2, buf_a, sem_sa)
            gwait(buf_b, sem_b)
            ascat(CPG - 1, buf_b, sem_sb)
            swait(buf_a, sem_sa)
            swait(buf_b, sem_sb)

        plsc.subcore_barrier()
        pltpu.sync_copy(acc_sh.at[pl.ds(sid * RPS, RPS)],
                        agg_hbm.at[cid].at[pl.ds(sid * RPS, RPS)])

    return pl.kernel(
        body, out_type=out_type, mesh=mesh, scratch_types=scratch,
        compiler_params=pltpu.CompilerParams(needs_layout_passes=False))


def _make_sc_deg():
    mesh = plsc.VectorSubcoreMesh(core_axis_name="c", subcore_axis_name="s")
    out_type = jax.ShapeDtypeStruct((NW * N,), jnp.float32)
    scratch = [
        pltpu.VMEM((DNCH, DCH), jnp.int32),
        pltpu.VMEM((N,), jnp.float32),
    ]

    def body(dst_hbm, deg_hbm, dst_v, deg_v):
        cid = lax.axis_index("c")
        sid = lax.axis_index("s")
        wid = cid * NS + sid

        pltpu.sync_copy(dst_hbm.at[wid], dst_v)
        zvec = jnp.zeros((LANES,), jnp.float32)

        def dz(i, carry):
            deg_v[pl.ds(i * LANES, LANES)] = zvec
            return carry

        lax.fori_loop(0, N // LANES, dz, 0)
        ones = jnp.ones((LANES,), jnp.float32)

        def dacc(i, carry):
            j = i // (DCH // LANES)
            k = (i % (DCH // LANES)) * LANES
            idx = dst_v[j, pl.ds(k, LANES)]
            plsc.addupdate_scatter(deg_v, [idx], ones)
            return carry

        lax.fori_loop(0, DNCH * (DCH // LANES), dacc, 0)
        pltpu.sync_copy(deg_v, deg_hbm.at[pl.ds(wid * N, N)])

    return pl.kernel(
        body, out_type=out_type, mesh=mesh, scratch_types=scratch,
        compiler_params=pltpu.CompilerParams(needs_layout_passes=False))


_sc_agg = _make_sc_agg()
_sc_deg = _make_sc_deg()

_TC_R = 1000


def _tc_layer1_body(agg_ref, x_ref, degt_ref, wt_ref, b_ref, out_ref, rinv_ref):
    agg = agg_ref[0] + agg_ref[1] + x_ref[...]
    deg = jnp.sum(degt_ref[...], axis=1, keepdims=True)
    rinv = 1.0 / (deg + 1.0)
    hn = agg * rinv
    y = jnp.dot(hn, wt_ref[...], preferred_element_type=jnp.float32) + b_ref[...]
    out_ref[...] = jax.nn.relu(y)
    rinv_ref[...] = jnp.broadcast_to(rinv, (_TC_R, D))


def _tc_layer_body(act, agg_ref, h_ref, rinv_ref, wt_ref, b_ref, out_ref):
    hn = (agg_ref[0] + agg_ref[1] + h_ref[...]) * rinv_ref[...]
    y = jnp.dot(hn, wt_ref[...], preferred_element_type=jnp.float32) + b_ref[...]
    out_ref[...] = act(y)


def _tc_layer1(aggp, x, degt, wt, b):
    grid = (N // _TC_R,)
    return pl.pallas_call(
        _tc_layer1_body,
        grid=grid,
        in_specs=[
            pl.BlockSpec((NC, _TC_R, D), lambda i: (0, i, 0)),
            pl.BlockSpec((_TC_R, D), lambda i: (i, 0)),
            pl.BlockSpec((_TC_R, NW), lambda i: (i, 0)),
            pl.BlockSpec((D, D), lambda i: (0, 0)),
            pl.BlockSpec((1, D), lambda i: (0, 0)),
        ],
        out_specs=[
            pl.BlockSpec((_TC_R, D), lambda i: (i, 0)),
            pl.BlockSpec((_TC_R, D), lambda i: (i, 0)),
        ],
        out_shape=[
            jax.ShapeDtypeStruct((N, D), jnp.float32),
            jax.ShapeDtypeStruct((N, D), jnp.float32),
        ],
    )(aggp, x, degt, wt, b)


def _tc_layer(aggp, h, rinv, wt, b, act):
    grid = (N // _TC_R,)
    return pl.pallas_call(
        functools.partial(_tc_layer_body, act),
        grid=grid,
        in_specs=[
            pl.BlockSpec((NC, _TC_R, D), lambda i: (0, i, 0)),
            pl.BlockSpec((_TC_R, D), lambda i: (i, 0)),
            pl.BlockSpec((_TC_R, D), lambda i: (i, 0)),
            pl.BlockSpec((D, D), lambda i: (0, 0)),
            pl.BlockSpec((1, D), lambda i: (0, 0)),
        ],
        out_specs=pl.BlockSpec((_TC_R, D), lambda i: (i, 0)),
        out_shape=jax.ShapeDtypeStruct((N, D), jnp.float32),
    )(aggp, h, rinv, wt, b)


def kernel(x, edge_index, W1, b1, W2, b2, W3, b3):
    npade = EPWP - EPW
    lane = jnp.arange(npade, dtype=jnp.int32)[None, :]
    wcol = jnp.arange(NW, dtype=jnp.int32)[:, None]
    pad_src = (wcol * npade + lane) % N
    pad_dst = N + (lane + wcol) % (NPAD - N)
    src = jnp.concatenate([edge_index[0].reshape(NW, EPW),
                           jnp.broadcast_to(pad_src, (NW, npade))],
                          axis=1).reshape(NW, NGRP, CPG, CH)
    dst = jnp.concatenate([edge_index[1].reshape(NW, EPW),
                           jnp.broadcast_to(pad_dst, (NW, npade))],
                          axis=1).reshape(NW, NGRP, CPG, CH)
    dst_flat = edge_index[1].reshape(NW, DNCH, DCH)

    degp = _sc_deg(dst_flat)
    aggp1 = _sc_agg(x, src, dst)
    degt = degp.reshape(NW, N).T
    h1, rinv = _tc_layer1(aggp1, x, degt, W1.T, b1.reshape(1, D))

    aggp2 = _sc_agg(h1, src, dst)
    h2 = _tc_layer(aggp2, h1, rinv, W2.T, b2.reshape(1, D), jax.nn.relu)

    aggp3 = _sc_agg(h2, src, dst)
    h3 = _tc_layer(aggp3, h2, rinv, W3.T, b3.reshape(1, D), jax.nn.sigmoid)
    return h3


# speedup vs baseline: 1.2234x; 1.2234x over previous
import functools

import jax
import jax.numpy as jnp
from jax import lax
from jax.experimental import pallas as pl
from jax.experimental.pallas import tpu as pltpu
from jax.experimental.pallas import tpu_sc as plsc

N = 10000
NPAD = 10240
D = 128
E = 320000
NC = 2
NS = 16
NW = NC * NS
EPW = E // NW
CH = 128
EPWP = 10240
NCHUNK = EPWP // CH
RPS = NPAD // NS
LANES = 16
DCH = 80
DNCH = EPW // DCH


NGRP = 5
CPG = NCHUNK // NGRP


def _make_sc_agg():
    mesh = plsc.VectorSubcoreMesh(core_axis_name="c", subcore_axis_name="s")
    out_type = jax.ShapeDtypeStruct((NC, NPAD, D), jnp.float32)
    scratch = [
        pltpu.VMEM((CPG, CH), jnp.int32),
        pltpu.VMEM((CPG, CH), jnp.int32),
        pltpu.VMEM((CH, D), jnp.float32),
        pltpu.VMEM((CH, D), jnp.float32),
        pltpu.VMEM_SHARED((NPAD, D), jnp.float32),
        pltpu.SemaphoreType.DMA,
        pltpu.SemaphoreType.DMA,
    ]

    def body(h_hbm, src_hbm, dst_hbm, agg_hbm,
             src_v, dst_v, buf_a, buf_b, acc_sh, sem_a, sem_b):
        cid = lax.axis_index("c")
        sid = lax.axis_index("s")
        wid = cid * NS + sid

        zvec = jnp.zeros((LANES,), jnp.float32)

        def zrow(i, carry):
            r = i // (D // LANES)
            c = (i % (D // LANES)) * LANES
            buf_a[r, pl.ds(c, LANES)] = zvec
            return carry

        lax.fori_loop(0, CH * (D // LANES), zrow, 0)
        for t in range(RPS // CH):
            pltpu.sync_copy(buf_a, acc_sh.at[pl.ds(sid * RPS + t * CH, CH)])

        plsc.subcore_barrier()

        def gather(j, buf, sem):
            return pltpu.async_copy(h_hbm.at[src_v.at[j]], buf, sem)

        def gwait(buf, sem):
            pltpu.make_async_copy(h_hbm.at[src_v.at[0]], buf, sem).wait()

        def scat(j, buf):
            pltpu.sync_copy(buf, acc_sh.at[dst_v.at[j]], add=True)

        for g in range(NGRP):
            pltpu.sync_copy(src_hbm.at[wid].at[g], src_v)
            pltpu.sync_copy(dst_hbm.at[wid].at[g], dst_v)
            gather(0, buf_a, sem_a)

            def pair(p, carry):
                j = 2 * p
                gather(j + 1, buf_b, sem_b)
                gwait(buf_a, sem_a)
                scat(j, buf_a)
                gather(j + 2, buf_a, sem_a)
                gwait(buf_b, sem_b)
                scat(j + 1, buf_b)
                return carry

            lax.fori_loop(0, CPG // 2 - 1, pair, 0)
            gather(CPG - 1, buf_b, sem_b)
            gwait(buf_a, sem_a)
            scat(CPG - 2, buf_a)
            gwait(buf_b, sem_b)
            scat(CPG - 1, buf_b)

        plsc.subcore_barrier()
        pltpu.sync_copy(acc_sh.at[pl.ds(sid * RPS, RPS)],
                        agg_hbm.at[cid].at[pl.ds(sid * RPS, RPS)])

    return pl.kernel(
        body, out_type=out_type, mesh=mesh, scratch_types=scratch,
        compiler_params=pltpu.CompilerParams(needs_layout_passes=False))


def _make_sc_deg():
    mesh = plsc.VectorSubcoreMesh(core_axis_name="c", subcore_axis_name="s")
    out_type = jax.ShapeDtypeStruct((NW * N,), jnp.float32)
    scratch = [
        pltpu.VMEM((DNCH, DCH), jnp.int32),
        pltpu.VMEM((N,), jnp.float32),
    ]

    def body(dst_hbm, deg_hbm, dst_v, deg_v):
        cid = lax.axis_index("c")
        sid = lax.axis_index("s")
        wid = cid * NS + sid

        pltpu.sync_copy(dst_hbm.at[wid], dst_v)
        zvec = jnp.zeros((LANES,), jnp.float32)

        def dz(i, carry):
            deg_v[pl.ds(i * LANES, LANES)] = zvec
            return carry

        lax.fori_loop(0, N // LANES, dz, 0)
        ones = jnp.ones((LANES,), jnp.float32)

        def dacc(i, carry):
            j = i // (DCH // LANES)
            k = (i % (DCH // LANES)) * LANES
            idx = dst_v[j, pl.ds(k, LANES)]
            plsc.addupdate_scatter(deg_v, [idx], ones)
            return carry

        lax.fori_loop(0, DNCH * (DCH // LANES), dacc, 0)
        pltpu.sync_copy(deg_v, deg_hbm.at[pl.ds(wid * N, N)])

    return pl.kernel(
        body, out_type=out_type, mesh=mesh, scratch_types=scratch,
        compiler_params=pltpu.CompilerParams(needs_layout_passes=False))


_sc_agg = _make_sc_agg()
_sc_deg = _make_sc_deg()

_TC_R = 1000


def _tc_layer1_body(agg_ref, x_ref, degt_ref, wt_ref, b_ref, out_ref, rinv_ref):
    agg = agg_ref[0] + agg_ref[1] + x_ref[...]
    deg = jnp.sum(degt_ref[...], axis=1, keepdims=True)
    rinv = 1.0 / (deg + 1.0)
    hn = agg * rinv
    y = jnp.dot(hn, wt_ref[...], preferred_element_type=jnp.float32) + b_ref[...]
    out_ref[...] = jax.nn.relu(y)
    rinv_ref[...] = jnp.broadcast_to(rinv, (_TC_R, D))


def _tc_layer_body(act, agg_ref, h_ref, rinv_ref, wt_ref, b_ref, out_ref):
    hn = (agg_ref[0] + agg_ref[1] + h_ref[...]) * rinv_ref[...]
    y = jnp.dot(hn, wt_ref[...], preferred_element_type=jnp.float32) + b_ref[...]
    out_ref[...] = act(y)


def _tc_layer1(aggp, x, degt, wt, b):
    grid = (N // _TC_R,)
    return pl.pallas_call(
        _tc_layer1_body,
        grid=grid,
        in_specs=[
            pl.BlockSpec((NC, _TC_R, D), lambda i: (0, i, 0)),
            pl.BlockSpec((_TC_R, D), lambda i: (i, 0)),
            pl.BlockSpec((_TC_R, NW), lambda i: (i, 0)),
            pl.BlockSpec((D, D), lambda i: (0, 0)),
            pl.BlockSpec((1, D), lambda i: (0, 0)),
        ],
        out_specs=[
            pl.BlockSpec((_TC_R, D), lambda i: (i, 0)),
            pl.BlockSpec((_TC_R, D), lambda i: (i, 0)),
        ],
        out_shape=[
            jax.ShapeDtypeStruct((N, D), jnp.float32),
            jax.ShapeDtypeStruct((N, D), jnp.float32),
        ],
    )(aggp, x, degt, wt, b)


def _tc_layer(aggp, h, rinv, wt, b, act):
    grid = (N // _TC_R,)
    return pl.pallas_call(
        functools.partial(_tc_layer_body, act),
        grid=grid,
        in_specs=[
            pl.BlockSpec((NC, _TC_R, D), lambda i: (0, i, 0)),
            pl.BlockSpec((_TC_R, D), lambda i: (i, 0)),
            pl.BlockSpec((_TC_R, D), lambda i: (i, 0)),
            pl.BlockSpec((D, D), lambda i: (0, 0)),
            pl.BlockSpec((1, D), lambda i: (0, 0)),
        ],
        out_specs=pl.BlockSpec((_TC_R, D), lambda i: (i, 0)),
        out_shape=jax.ShapeDtypeStruct((N, D), jnp.float32),
    )(aggp, h, rinv, wt, b)


def kernel(x, edge_index, W1, b1, W2, b2, W3, b3):
    npade = EPWP - EPW
    lane = jnp.arange(npade, dtype=jnp.int32)[None, :]
    wcol = jnp.arange(NW, dtype=jnp.int32)[:, None]
    pad_src = (wcol * npade + lane) % N
    pad_dst = N + (lane + wcol) % (NPAD - N)
    src = jnp.concatenate([edge_index[0].reshape(NW, EPW),
                           jnp.broadcast_to(pad_src, (NW, npade))],
                          axis=1).reshape(NW, NGRP, CPG, CH)
    dst = jnp.concatenate([edge_index[1].reshape(NW, EPW),
                           jnp.broadcast_to(pad_dst, (NW, npade))],
                          axis=1).reshape(NW, NGRP, CPG, CH)
    dst_flat = edge_index[1].reshape(NW, DNCH, DCH)

    degp = _sc_deg(dst_flat)
    aggp1 = _sc_agg(x, src, dst)
    degt = degp.reshape(NW, N).T
    h1, rinv = _tc_layer1(aggp1, x, degt, W1.T, b1.reshape(1, D))

    aggp2 = _sc_agg(h1, src, dst)
    h2 = _tc_layer(aggp2, h1, rinv, W2.T, b2.reshape(1, D), jax.nn.relu)

    aggp3 = _sc_agg(h2, src, dst)
    h3 = _tc_layer(aggp3, h2, rinv, W3.T, b3.reshape(1, D), jax.nn.sigmoid)
    return h3


# speedup vs baseline: 1.2467x; 1.0190x over previous
import functools

import jax
import jax.numpy as jnp
from jax import lax
from jax.experimental import pallas as pl
from jax.experimental.pallas import tpu as pltpu
from jax.experimental.pallas import tpu_sc as plsc

N = 10000
NPAD = 10240
D = 128
E = 320000
NC = 2
NS = 16
NW = NC * NS
EPW = E // NW
CH = 128
EPWP = 10240
NCHUNK = EPWP // CH
RPS = NPAD // NS
LANES = 16
DCH = 80
DNCH = EPW // DCH


NGRP = 4
CPG = NCHUNK // NGRP


def _make_sc_agg():
    mesh = plsc.VectorSubcoreMesh(core_axis_name="c", subcore_axis_name="s")
    out_type = jax.ShapeDtypeStruct((NC, NPAD, D), jnp.float32)
    scratch = [
        pltpu.VMEM((CPG, CH), jnp.int32),
        pltpu.VMEM((CPG, CH), jnp.int32),
        pltpu.VMEM((CH, D), jnp.float32),
        pltpu.VMEM((CH, D), jnp.float32),
        pltpu.VMEM_SHARED((NPAD, D), jnp.float32),
        pltpu.SemaphoreType.DMA,
        pltpu.SemaphoreType.DMA,
    ]

    def body(h_hbm, src_hbm, dst_hbm, agg_hbm,
             src_v, dst_v, buf_a, buf_b, acc_sh, sem_a, sem_b):
        cid = lax.axis_index("c")
        sid = lax.axis_index("s")
        wid = cid * NS + sid

        zvec = jnp.zeros((LANES,), jnp.float32)

        def zrow(i, carry):
            r = i // (D // LANES)
            c = (i % (D // LANES)) * LANES
            buf_a[r, pl.ds(c, LANES)] = zvec
            return carry

        lax.fori_loop(0, CH * (D // LANES), zrow, 0)
        for t in range(RPS // CH):
            pltpu.sync_copy(buf_a, acc_sh.at[pl.ds(sid * RPS + t * CH, CH)])

        plsc.subcore_barrier()

        def gather(j, buf, sem):
            return pltpu.async_copy(h_hbm.at[src_v.at[j]], buf, sem)

        def gwait(buf, sem):
            pltpu.make_async_copy(h_hbm.at[src_v.at[0]], buf, sem).wait()

        def scat(j, buf):
            pltpu.sync_copy(buf, acc_sh.at[dst_v.at[j]], add=True)

        for g in range(NGRP):
            pltpu.sync_copy(src_hbm.at[wid].at[g], src_v)
            pltpu.sync_copy(dst_hbm.at[wid].at[g], dst_v)
            gather(0, buf_a, sem_a)

            def pair(p, carry):
                j = 2 * p
                gather(j + 1, buf_b, sem_b)
                gwait(buf_a, sem_a)
                scat(j, buf_a)
                gather(j + 2, buf_a, sem_a)
                gwait(buf_b, sem_b)
                scat(j + 1, buf_b)
                return carry

            lax.fori_loop(0, CPG // 2 - 1, pair, 0)
            gather(CPG - 1, buf_b, sem_b)
            gwait(buf_a, sem_a)
            scat(CPG - 2, buf_a)
            gwait(buf_b, sem_b)
            scat(CPG - 1, buf_b)

        plsc.subcore_barrier()
        pltpu.sync_copy(acc_sh.at[pl.ds(sid * RPS, RPS)],
                        agg_hbm.at[cid].at[pl.ds(sid * RPS, RPS)])

    return pl.kernel(
        body, out_type=out_type, mesh=mesh, scratch_types=scratch,
        compiler_params=pltpu.CompilerParams(needs_layout_passes=False))


def _make_sc_deg():
    mesh = plsc.VectorSubcoreMesh(core_axis_name="c", subcore_axis_name="s")
    out_type = jax.ShapeDtypeStruct((NW * N,), jnp.float32)
    scratch = [
        pltpu.VMEM((DNCH, DCH), jnp.int32),
        pltpu.VMEM((N,), jnp.float32),
    ]

    def body(dst_hbm, deg_hbm, dst_v, deg_v):
        cid = lax.axis_index("c")
        sid = lax.axis_index("s")
        wid = cid * NS + sid

        pltpu.sync_copy(dst_hbm.at[wid], dst_v)
        zvec = jnp.zeros((LANES,), jnp.float32)

        def dz(i, carry):
            deg_v[pl.ds(i * LANES, LANES)] = zvec
            return carry

        lax.fori_loop(0, N // LANES, dz, 0)
        ones = jnp.ones((LANES,), jnp.float32)

        def dacc(i, carry):
            j = i // (DCH // LANES)
            k = (i % (DCH // LANES)) * LANES
            idx = dst_v[j, pl.ds(k, LANES)]
            plsc.addupdate_scatter(deg_v, [idx], ones)
            return carry

        lax.fori_loop(0, DNCH * (DCH // LANES), dacc, 0)
        pltpu.sync_copy(deg_v, deg_hbm.at[pl.ds(wid * N, N)])

    return pl.kernel(
        body, out_type=out_type, mesh=mesh, scratch_types=scratch,
        compiler_params=pltpu.CompilerParams(needs_layout_passes=False))


_sc_agg = _make_sc_agg()
_sc_deg = _make_sc_deg()

_TC_R = 1000


def _tc_layer1_body(agg_ref, x_ref, degt_ref, wt_ref, b_ref, out_ref, rinv_ref):
    agg = agg_ref[0] + agg_ref[1] + x_ref[...]
    deg = jnp.sum(degt_ref[...], axis=1, keepdims=True)
    rinv = 1.0 / (deg + 1.0)
    hn = agg * rinv
    y = jnp.dot(hn, wt_ref[...], preferred_element_type=jnp.float32) + b_ref[...]
    out_ref[...] = jax.nn.relu(y)
    rinv_ref[...] = jnp.broadcast_to(rinv, (_TC_R, D))


def _tc_layer_body(act, agg_ref, h_ref, rinv_ref, wt_ref, b_ref, out_ref):
    hn = (agg_ref[0] + agg_ref[1] + h_ref[...]) * rinv_ref[...]
    y = jnp.dot(hn, wt_ref[...], preferred_element_type=jnp.float32) + b_ref[...]
    out_ref[...] = act(y)


def _tc_layer1(aggp, x, degt, wt, b):
    grid = (N // _TC_R,)
    return pl.pallas_call(
        _tc_layer1_body,
        grid=grid,
        in_specs=[
            pl.BlockSpec((NC, _TC_R, D), lambda i: (0, i, 0)),
            pl.BlockSpec((_TC_R, D), lambda i: (i, 0)),
            pl.BlockSpec((_TC_R, NW), lambda i: (i, 0)),
            pl.BlockSpec((D, D), lambda i: (0, 0)),
            pl.BlockSpec((1, D), lambda i: (0, 0)),
        ],
        out_specs=[
            pl.BlockSpec((_TC_R, D), lambda i: (i, 0)),
            pl.BlockSpec((_TC_R, D), lambda i: (i, 0)),
        ],
        out_shape=[
            jax.ShapeDtypeStruct((N, D), jnp.float32),
            jax.ShapeDtypeStruct((N, D), jnp.float32),
        ],
    )(aggp, x, degt, wt, b)


def _tc_layer(aggp, h, rinv, wt, b, act):
    grid = (N // _TC_R,)
    return pl.pallas_call(
        functools.partial(_tc_layer_body, act),
        grid=grid,
        in_specs=[
            pl.BlockSpec((NC, _TC_R, D), lambda i: (0, i, 0)),
            pl.BlockSpec((_TC_R, D), lambda i: (i, 0)),
            pl.BlockSpec((_TC_R, D), lambda i: (i, 0)),
            pl.BlockSpec((D, D), lambda i: (0, 0)),
            pl.BlockSpec((1, D), lambda i: (0, 0)),
        ],
        out_specs=pl.BlockSpec((_TC_R, D), lambda i: (i, 0)),
        out_shape=jax.ShapeDtypeStruct((N, D), jnp.float32),
    )(aggp, h, rinv, wt, b)


def kernel(x, edge_index, W1, b1, W2, b2, W3, b3):
    npade = EPWP - EPW
    lane = jnp.arange(npade, dtype=jnp.int32)[None, :]
    wcol = jnp.arange(NW, dtype=jnp.int32)[:, None]
    pad_src = (wcol * npade + lane) % N
    pad_dst = N + (lane + wcol) % (NPAD - N)
    src = jnp.concatenate([edge_index[0].reshape(NW, EPW),
                           jnp.broadcast_to(pad_src, (NW, npade))],
                          axis=1).reshape(NW, NGRP, CPG, CH)
    dst = jnp.concatenate([edge_index[1].reshape(NW, EPW),
                           jnp.broadcast_to(pad_dst, (NW, npade))],
                          axis=1).reshape(NW, NGRP, CPG, CH)
    dst_flat = edge_index[1].reshape(NW, DNCH, DCH)

    degp = _sc_deg(dst_flat)
    aggp1 = _sc_agg(x, src, dst)
    degt = degp.reshape(NW, N).T
    h1, rinv = _tc_layer1(aggp1, x, degt, W1.T, b1.reshape(1, D))

    aggp2 = _sc_agg(h1, src, dst)
    h2 = _tc_layer(aggp2, h1, rinv, W2.T, b2.reshape(1, D), jax.nn.relu)

    aggp3 = _sc_agg(h2, src, dst)
    h3 = _tc_layer(aggp3, h2, rinv, W3.T, b3.reshape(1, D), jax.nn.sigmoid)
    return h3


# speedup vs baseline: 1.3016x; 1.0440x over previous
import functools

import jax
import jax.numpy as jnp
from jax import lax
from jax.experimental import pallas as pl
from jax.experimental.pallas import tpu as pltpu
from jax.experimental.pallas import tpu_sc as plsc

N = 10000
NPAD = 10240
D = 128
E = 320000
NC = 2
NS = 16
NW = NC * NS
EPW = E // NW
CH = 128
EPWP = 10240
NCHUNK = EPWP // CH
RPS = NPAD // NS
LANES = 16
DCH = 80
DNCH = EPW // DCH


NGRP = 2
CPG = NCHUNK // NGRP


def _make_sc_agg():
    mesh = plsc.VectorSubcoreMesh(core_axis_name="c", subcore_axis_name="s")
    out_type = jax.ShapeDtypeStruct((NC, NPAD, D), jnp.float32)
    scratch = [
        pltpu.VMEM((CPG, CH), jnp.int32),
        pltpu.VMEM((CPG, CH), jnp.int32),
        pltpu.VMEM((CH, D), jnp.float32),
        pltpu.VMEM((CH, D), jnp.float32),
        pltpu.VMEM_SHARED((NPAD, D), jnp.float32),
        pltpu.SemaphoreType.DMA,
        pltpu.SemaphoreType.DMA,
    ]

    def body(h_hbm, src_hbm, dst_hbm, agg_hbm,
             src_v, dst_v, buf_a, buf_b, acc_sh, sem_a, sem_b):
        cid = lax.axis_index("c")
        sid = lax.axis_index("s")
        wid = cid * NS + sid

        zvec = jnp.zeros((LANES,), jnp.float32)

        def zrow(i, carry):
            r = i // (D // LANES)
            c = (i % (D // LANES)) * LANES
            buf_a[r, pl.ds(c, LANES)] = zvec
            return carry

        lax.fori_loop(0, CH * (D // LANES), zrow, 0)
        for t in range(RPS // CH):
            pltpu.sync_copy(buf_a, acc_sh.at[pl.ds(sid * RPS + t * CH, CH)])

        plsc.subcore_barrier()

        def gather(j, buf, sem):
            return pltpu.async_copy(h_hbm.at[src_v.at[j]], buf, sem)

        def gwait(buf, sem):
            pltpu.make_async_copy(h_hbm.at[src_v.at[0]], buf, sem).wait()

        def scat(j, buf):
            pltpu.sync_copy(buf, acc_sh.at[dst_v.at[j]], add=True)

        for g in range(NGRP):
            pltpu.sync_copy(src_hbm.at[wid].at[g], src_v)
            pltpu.sync_copy(dst_hbm.at[wid].at[g], dst_v)
            gather(0, buf_a, sem_a)

            def pair(p, carry):
                j = 2 * p
                gather(j + 1, buf_b, sem_b)
                gwait(buf_a, sem_a)
                scat(j, buf_a)
                gather(j + 2, buf_a, sem_a)
                gwait(buf_b, sem_b)
                scat(j + 1, buf_b)
                return carry

            lax.fori_loop(0, CPG // 2 - 1, pair, 0)
            gather(CPG - 1, buf_b, sem_b)
            gwait(buf_a, sem_a)
            scat(CPG - 2, buf_a)
            gwait(buf_b, sem_b)
            scat(CPG - 1, buf_b)

        plsc.subcore_barrier()
        pltpu.sync_copy(acc_sh.at[pl.ds(sid * RPS, RPS)],
                        agg_hbm.at[cid].at[pl.ds(sid * RPS, RPS)])

    return pl.kernel(
        body, out_type=out_type, mesh=mesh, scratch_types=scratch,
        compiler_params=pltpu.CompilerParams(needs_layout_passes=False))


def _make_sc_deg():
    mesh = plsc.VectorSubcoreMesh(core_axis_name="c", subcore_axis_name="s")
    out_type = jax.ShapeDtypeStruct((NW * N,), jnp.float32)
    scratch = [
        pltpu.VMEM((DNCH, DCH), jnp.int32),
        pltpu.VMEM((N,), jnp.float32),
    ]

    def body(dst_hbm, deg_hbm, dst_v, deg_v):
        cid = lax.axis_index("c")
        sid = lax.axis_index("s")
        wid = cid * NS + sid

        pltpu.sync_copy(dst_hbm.at[wid], dst_v)
        zvec = jnp.zeros((LANES,), jnp.float32)

        def dz(i, carry):
            deg_v[pl.ds(i * LANES, LANES)] = zvec
            return carry

        lax.fori_loop(0, N // LANES, dz, 0)
        ones = jnp.ones((LANES,), jnp.float32)

        def dacc(i, carry):
            j = i // (DCH // LANES)
            k = (i % (DCH // LANES)) * LANES
            idx = dst_v[j, pl.ds(k, LANES)]
            plsc.addupdate_scatter(deg_v, [idx], ones)
            return carry

        lax.fori_loop(0, DNCH * (DCH // LANES), dacc, 0)
        pltpu.sync_copy(deg_v, deg_hbm.at[pl.ds(wid * N, N)])

    return pl.kernel(
        body, out_type=out_type, mesh=mesh, scratch_types=scratch,
        compiler_params=pltpu.CompilerParams(needs_layout_passes=False))


_sc_agg = _make_sc_agg()
_sc_deg = _make_sc_deg()

_TC_R = 1000


def _tc_layer1_body(agg_ref, x_ref, degt_ref, wt_ref, b_ref, out_ref, rinv_ref):
    agg = agg_ref[0] + agg_ref[1] + x_ref[...]
    deg = jnp.sum(degt_ref[...], axis=1, keepdims=True)
    rinv = 1.0 / (deg + 1.0)
    hn = agg * rinv
    y = jnp.dot(hn, wt_ref[...], preferred_element_type=jnp.float32) + b_ref[...]
    out_ref[...] = jax.nn.relu(y)
    rinv_ref[...] = jnp.broadcast_to(rinv, (_TC_R, D))


def _tc_layer_body(act, agg_ref, h_ref, rinv_ref, wt_ref, b_ref, out_ref):
    hn = (agg_ref[0] + agg_ref[1] + h_ref[...]) * rinv_ref[...]
    y = jnp.dot(hn, wt_ref[...], preferred_element_type=jnp.float32) + b_ref[...]
    out_ref[...] = act(y)


def _tc_layer1(aggp, x, degt, wt, b):
    grid = (N // _TC_R,)
    return pl.pallas_call(
        _tc_layer1_body,
        grid=grid,
        in_specs=[
            pl.BlockSpec((NC, _TC_R, D), lambda i: (0, i, 0)),
            pl.BlockSpec((_TC_R, D), lambda i: (i, 0)),
            pl.BlockSpec((_TC_R, NW), lambda i: (i, 0)),
            pl.BlockSpec((D, D), lambda i: (0, 0)),
            pl.BlockSpec((1, D), lambda i: (0, 0)),
        ],
        out_specs=[
            pl.BlockSpec((_TC_R, D), lambda i: (i, 0)),
            pl.BlockSpec((_TC_R, D), lambda i: (i, 0)),
        ],
        out_shape=[
            jax.ShapeDtypeStruct((N, D), jnp.float32),
            jax.ShapeDtypeStruct((N, D), jnp.float32),
        ],
    )(aggp, x, degt, wt, b)


def _tc_layer(aggp, h, rinv, wt, b, act):
    grid = (N // _TC_R,)
    return pl.pallas_call(
        functools.partial(_tc_layer_body, act),
        grid=grid,
        in_specs=[
            pl.BlockSpec((NC, _TC_R, D), lambda i: (0, i, 0)),
            pl.BlockSpec((_TC_R, D), lambda i: (i, 0)),
            pl.BlockSpec((_TC_R, D), lambda i: (i, 0)),
            pl.BlockSpec((D, D), lambda i: (0, 0)),
            pl.BlockSpec((1, D), lambda i: (0, 0)),
        ],
        out_specs=pl.BlockSpec((_TC_R, D), lambda i: (i, 0)),
        out_shape=jax.ShapeDtypeStruct((N, D), jnp.float32),
    )(aggp, h, rinv, wt, b)


def kernel(x, edge_index, W1, b1, W2, b2, W3, b3):
    npade = EPWP - EPW
    lane = jnp.arange(npade, dtype=jnp.int32)[None, :]
    wcol = jnp.arange(NW, dtype=jnp.int32)[:, None]
    pad_src = (wcol * npade + lane) % N
    pad_dst = N + (lane + wcol) % (NPAD - N)
    src = jnp.concatenate([edge_index[0].reshape(NW, EPW),
                           jnp.broadcast_to(pad_src, (NW, npade))],
                          axis=1).reshape(NW, NGRP, CPG, CH)
    dst = jnp.concatenate([edge_index[1].reshape(NW, EPW),
                           jnp.broadcast_to(pad_dst, (NW, npade))],
                          axis=1).reshape(NW, NGRP, CPG, CH)
    dst_flat = edge_index[1].reshape(NW, DNCH, DCH)

    degp = _sc_deg(dst_flat)
    aggp1 = _sc_agg(x, src, dst)
    degt = degp.reshape(NW, N).T
    h1, rinv = _tc_layer1(aggp1, x, degt, W1.T, b1.reshape(1, D))

    aggp2 = _sc_agg(h1, src, dst)
    h2 = _tc_layer(aggp2, h1, rinv, W2.T, b2.reshape(1, D), jax.nn.relu)

    aggp3 = _sc_agg(h2, src, dst)
    h3 = _tc_layer(aggp3, h2, rinv, W3.T, b3.reshape(1, D), jax.nn.sigmoid)
    return h3
